# baseline (speedup 1.0000x reference)
import jax
import jax.numpy as jnp
from jax import lax
from jax.experimental import pallas as pl
from jax.experimental.pallas import tpu as pltpu
from jax.experimental.pallas import tpu_sc as plsc

B = 4096
LT = 20
LD = 200
D = 32
H = 16
NC = 2
NS = 16
NW = NC * NS
RPW = B // NW
NP = RPW // 2
LPAIR = 2 * LT + 2 * LD
DO0 = 2 * LT
DO1 = 2 * LT + LD


def _pool(rbuf, off, L):
    ninf = jnp.full((H,), -jnp.inf, jnp.float32)
    halves = []
    for h in (0, H):
        x0 = rbuf[off + 0, pl.ds(h, H)]
        x1 = rbuf[off + 1, pl.ds(h, H)]
        x2 = rbuf[off + 2, pl.ds(h, H)]
        x3 = rbuf[off + 3, pl.ds(h, H)]
        halves.append((x0 + x1, x1 + x2, x2 + x3, x2, x3, ninf, ninf))

    def upd(x, w0, w1, w2, x2, x3, m3, m5):
        w3 = x3 + x[0]
        w4 = x[0] + x[1]
        w5 = x[1] + x[2]
        w6 = x[2] + x[3]
        s30 = w0 + x2
        s31 = w1 + x3
        s32 = w2 + x[0]
        s33 = w3 + x[1]
        s50 = s30 + w3
        s51 = s31 + w4
        s52 = s32 + w5
        s53 = s33 + w6
        m3 = jnp.maximum(m3, jnp.maximum(jnp.maximum(s30, s31),
                                         jnp.maximum(s32, s33)))
        m5 = jnp.maximum(m5, jnp.maximum(jnp.maximum(s50, s51),
                                         jnp.maximum(s52, s53)))
        return (w4, w5, w6, x[2], x[3], m3, m5)

    def body8(i, c):
        base = off + 4 + i * 8
        xa = [rbuf[base + t, pl.ds(0, H)] for t in range(8)]
        xb = [rbuf[base + t, pl.ds(H, H)] for t in range(8)]
        ca = upd(xa[4:], *upd(xa[:4], *c[:7]))
        cb = upd(xb[4:], *upd(xb[:4], *c[7:]))
        return ca + cb

    n8 = (L - 4) // 8
    res = lax.fori_loop(0, n8, body8, halves[0] + halves[1])
    if (L - 4) % 8:
        base = off + 4 + n8 * 8
        xa = [rbuf[base + t, pl.ds(0, H)] for t in range(4)]
        xb = [rbuf[base + t, pl.ds(H, H)] for t in range(4)]
        res = upd(xa, *res[:7]) + upd(xb, *res[7:])

    ms = []
    for w0, w1, _w2, x2, x3, m3, m5 in (res[:7], res[7:]):
        ms.append((jnp.maximum(m3, jnp.maximum(w0 + x2, w1 + x3)), m5))
    return ms[0][0], ms[1][0], ms[0][1], ms[1][1]


def _sc_body(title_hbm, desc_hbm, table_hbm, out_hbm,
             tidx, didx, rbuf0, rbuf1, rbuf2, rbuf3, obuf,
             sem0, sem1, sem2, sem3):
    cid = lax.axis_index("c")
    sid = lax.axis_index("s")
    wid = sid * NC + cid
    base = wid * RPW

    pltpu.sync_copy(title_hbm.at[pl.ds(base * LT, RPW * LT)], tidx)
    pltpu.sync_copy(desc_hbm.at[pl.ds(base * LD, RPW * LD)], didx)

    def issue(p, rbuf, sem):
        pltpu.async_copy(table_hbm.at[tidx.at[pl.ds(p * 2 * LT, 2 * LT)]],
                         rbuf.at[pl.ds(0, 2 * LT)], sem)
        pltpu.async_copy(table_hbm.at[didx.at[pl.ds(p * 2 * LD, LD)]],
                         rbuf.at[pl.ds(DO0, LD)], sem)
        pltpu.async_copy(table_hbm.at[didx.at[pl.ds(p * 2 * LD + LD, LD)]],
                         rbuf.at[pl.ds(DO1, LD)], sem)

    def drain(rbuf, sem):
        pltpu.make_async_copy(table_hbm.at[pl.ds(0, LPAIR)], rbuf, sem).wait()

    def compute(p, rbuf):
        for r, toff, doff in ((0, 0, DO0), (1, LT, DO1)):
            b = p * 2 + r
            c3 = jnp.float32(1.0 / 3.0)
            c5 = jnp.float32(0.2)
            t3l, t3h, t5l, t5h = _pool(rbuf, toff, LT)
            d3l, d3h, d5l, d5h = _pool(rbuf, doff, LD)
            obuf[b, pl.ds(0, H)] = t3l * c3
            obuf[b, pl.ds(H, H)] = t3h * c3
            obuf[b, pl.ds(D, H)] = d3l * c3
            obuf[b, pl.ds(D + H, H)] = d3h * c3
            obuf[b, pl.ds(2 * D, H)] = t5l * c5
            obuf[b, pl.ds(2 * D + H, H)] = t5h * c5
            obuf[b, pl.ds(3 * D, H)] = d5l * c5
            obuf[b, pl.ds(3 * D + H, H)] = d5h * c5

    issue(0, rbuf0, sem0)
    issue(1, rbuf1, sem1)
    issue(2, rbuf2, sem2)

    def quad(i, carry):
        p0 = i * 4
        for k, (rb, sem) in enumerate(((rbuf0, sem0), (rbuf1, sem1),
                                       (rbuf2, sem2), (rbuf3, sem3))):
            nxt = ((rbuf3, sem3), (rbuf0, sem0),
                   (rbuf1, sem1), (rbuf2, sem2))[k]
            @pl.when(p0 + k + 3 < NP)
            def _(p=p0 + k + 3, n=nxt):
                issue(p, n[0], n[1])

            drain(rb, sem)
            compute(p0 + k, rb)
        return carry

    lax.fori_loop(0, NP // 4, quad, 0)

    pltpu.sync_copy(obuf, out_hbm.at[pl.ds(base, RPW)])


@jax.jit
def kernel(title, desc, t_len, d_len, table):
    del t_len, d_len
    mesh = plsc.VectorSubcoreMesh(core_axis_name="c", subcore_axis_name="s")
    run = pl.kernel(
        _sc_body,
        mesh=mesh,
        compiler_params=pltpu.CompilerParams(use_tc_tiling_on_sc=False),
        out_type=jax.ShapeDtypeStruct((B, 4 * D), jnp.float32),
        scratch_types=[
            pltpu.VMEM((RPW * LT,), jnp.int32),
            pltpu.VMEM((RPW * LD,), jnp.int32),
            pltpu.VMEM((LPAIR, D), jnp.float32),
            pltpu.VMEM((LPAIR, D), jnp.float32),
            pltpu.VMEM((LPAIR, D), jnp.float32),
            pltpu.VMEM((LPAIR, D), jnp.float32),
            pltpu.VMEM((RPW, 4 * D), jnp.float32),
            pltpu.SemaphoreType.DMA,
            pltpu.SemaphoreType.DMA,
            pltpu.SemaphoreType.DMA,
            pltpu.SemaphoreType.DMA,
        ],
    )
    return run(title.reshape(-1), desc.reshape(-1), table)


# speedup vs baseline: 1.0045x; 1.0045x over previous
import jax
import jax.numpy as jnp
from jax import lax
from jax.experimental import pallas as pl
from jax.experimental.pallas import tpu as pltpu
from jax.experimental.pallas import tpu_sc as plsc

B = 4096
LT = 20
LD = 200
D = 32
H = 16
NC = 2
NS = 16
NW = NC * NS
RPW = B // NW
NP = RPW // 2
LPAIR = 2 * LT + 2 * LD
DO0 = 2 * LT
DO1 = 2 * LT + LD


def _pool(rbuf, off, L):
    ninf = jnp.full((H,), -jnp.inf, jnp.float32)
    halves = []
    for h in (0, H):
        x0 = rbuf[off + 0, pl.ds(h, H)]
        x1 = rbuf[off + 1, pl.ds(h, H)]
        x2 = rbuf[off + 2, pl.ds(h, H)]
        x3 = rbuf[off + 3, pl.ds(h, H)]
        halves.append((x0 + x1, x1 + x2, x2 + x3, x2, x3, ninf, ninf))

    def upd(x, w0, w1, w2, x2, x3, m3, m5):
        w3 = x3 + x[0]
        w4 = x[0] + x[1]
        w5 = x[1] + x[2]
        w6 = x[2] + x[3]
        s30 = w0 + x2
        s31 = w1 + x3
        s32 = w2 + x[0]
        s33 = w3 + x[1]
        s50 = s30 + w3
        s51 = s31 + w4
        s52 = s32 + w5
        s53 = s33 + w6
        m3 = jnp.maximum(m3, jnp.maximum(jnp.maximum(s30, s31),
                                         jnp.maximum(s32, s33)))
        m5 = jnp.maximum(m5, jnp.maximum(jnp.maximum(s50, s51),
                                         jnp.maximum(s52, s53)))
        return (w4, w5, w6, x[2], x[3], m3, m5)

    def body8(i, c):
        base = off + 4 + i * 8
        xa = [rbuf[base + t, pl.ds(0, H)] for t in range(8)]
        xb = [rbuf[base + t, pl.ds(H, H)] for t in range(8)]
        ca = upd(xa[4:], *upd(xa[:4], *c[:7]))
        cb = upd(xb[4:], *upd(xb[:4], *c[7:]))
        return ca + cb

    n8 = (L - 4) // 8
    res = lax.fori_loop(0, n8, body8, halves[0] + halves[1])
    if (L - 4) % 8:
        base = off + 4 + n8 * 8
        xa = [rbuf[base + t, pl.ds(0, H)] for t in range(4)]
        xb = [rbuf[base + t, pl.ds(H, H)] for t in range(4)]
        res = upd(xa, *res[:7]) + upd(xb, *res[7:])

    ms = []
    for w0, w1, _w2, x2, x3, m3, m5 in (res[:7], res[7:]):
        ms.append((jnp.maximum(m3, jnp.maximum(w0 + x2, w1 + x3)), m5))
    return ms[0][0], ms[1][0], ms[0][1], ms[1][1]


def _sc_body(title_hbm, desc_hbm, table_hbm, out_hbm,
             tidx, didx, rbuf0, rbuf1, rbuf2, rbuf3, obuf,
             sem0, sem1, sem2, sem3):
    cid = lax.axis_index("c")
    sid = lax.axis_index("s")
    wid = sid * NC + cid
    base = wid * RPW

    pltpu.sync_copy(title_hbm.at[pl.ds(base * LT, RPW * LT)], tidx)
    pltpu.sync_copy(desc_hbm.at[pl.ds(base * LD, RPW * LD)], didx)

    def issue(p, rbuf, sem):
        pltpu.async_copy(table_hbm.at[tidx.at[pl.ds(p * 2 * LT, 2 * LT)]],
                         rbuf.at[pl.ds(0, 2 * LT)], sem)
        pltpu.async_copy(table_hbm.at[didx.at[pl.ds(p * 2 * LD, LD)]],
                         rbuf.at[pl.ds(DO0, LD)], sem)
        pltpu.async_copy(table_hbm.at[didx.at[pl.ds(p * 2 * LD + LD, LD)]],
                         rbuf.at[pl.ds(DO1, LD)], sem)

    def drain(rbuf, sem):
        pltpu.make_async_copy(table_hbm.at[pl.ds(0, LPAIR)], rbuf, sem).wait()

    def compute(p, rbuf):
        for r, toff, doff in ((0, 0, DO0), (1, LT, DO1)):
            b = p * 2 + r
            c3 = jnp.float32(1.0 / 3.0)
            c5 = jnp.float32(0.2)
            t3l, t3h, t5l, t5h = _pool(rbuf, toff, LT)
            d3l, d3h, d5l, d5h = _pool(rbuf, doff, LD)
            obuf[b, pl.ds(0, H)] = t3l * c3
            obuf[b, pl.ds(H, H)] = t3h * c3
            obuf[b, pl.ds(D, H)] = d3l * c3
            obuf[b, pl.ds(D + H, H)] = d3h * c3
            obuf[b, pl.ds(2 * D, H)] = t5l * c5
            obuf[b, pl.ds(2 * D + H, H)] = t5h * c5
            obuf[b, pl.ds(3 * D, H)] = d5l * c5
            obuf[b, pl.ds(3 * D + H, H)] = d5h * c5

    issue(0, rbuf0, sem0)
    issue(1, rbuf1, sem1)
    issue(2, rbuf2, sem2)

    def quad(i, carry):
        p0 = i * 4
        for k, (rb, sem) in enumerate(((rbuf0, sem0), (rbuf1, sem1),
                                       (rbuf2, sem2), (rbuf3, sem3))):
            nxt = ((rbuf3, sem3), (rbuf0, sem0),
                   (rbuf1, sem1), (rbuf2, sem2))[k]
            drain(rb, sem)

            @pl.when(p0 + k + 3 < NP)
            def _(p=p0 + k + 3, n=nxt):
                issue(p, n[0], n[1])

            compute(p0 + k, rb)
        return carry

    lax.fori_loop(0, NP // 4, quad, 0)

    pltpu.sync_copy(obuf, out_hbm.at[pl.ds(base, RPW)])


@jax.jit
def kernel(title, desc, t_len, d_len, table):
    del t_len, d_len
    mesh = plsc.VectorSubcoreMesh(core_axis_name="c", subcore_axis_name="s")
    run = pl.kernel(
        _sc_body,
        mesh=mesh,
        compiler_params=pltpu.CompilerParams(use_tc_tiling_on_sc=False),
        out_type=jax.ShapeDtypeStruct((B, 4 * D), jnp.float32),
        scratch_types=[
            pltpu.VMEM((RPW * LT,), jnp.int32),
            pltpu.VMEM((RPW * LD,), jnp.int32),
            pltpu.VMEM((LPAIR, D), jnp.float32),
            pltpu.VMEM((LPAIR, D), jnp.float32),
            pltpu.VMEM((LPAIR, D), jnp.float32),
            pltpu.VMEM((LPAIR, D), jnp.float32),
            pltpu.VMEM((RPW, 4 * D), jnp.float32),
            pltpu.SemaphoreType.DMA,
            pltpu.SemaphoreType.DMA,
            pltpu.SemaphoreType.DMA,
            pltpu.SemaphoreType.DMA,
        ],
    )
    return run(title.reshape(-1), desc.reshape(-1), table)
